# baseline (speedup 1.0000x reference)
import jax
import jax.numpy as jnp
from jax.experimental import pallas as pl
from jax.experimental.pallas import tpu as pltpu

BS = 1024
SC = 2 * BS // 128
TC = BS // 128
F32 = jnp.float32
BF16 = jnp.bfloat16


def _sides(mv, svt_j, sb_i):
    ga = jax.lax.dot_general(
        mv, svt_j, (((2,), (1,)), ((0,), (0,))),
        preferred_element_type=F32)
    a_side = jnp.sum(ga, axis=0)
    gb = jax.lax.dot_general(
        mv, sb_i, (((1,), (0,)), ((), ())),
        preferred_element_type=F32)
    b_side = gb.reshape(BS, 128)
    return a_side, b_side


def _pass1_body(a_ref, wv_ref, p_ref, x_ref, w0_ref, b0_ref,
                out0_ref, mv_ref, s0_scr, svt_scr):
    i = pl.program_id(0)
    j = pl.program_id(1)
    n = out0_ref.shape[0]

    @pl.when(jnp.logical_and(i == 0, j == 0))
    def _init():
        s0 = jnp.dot(x_ref[...].astype(BF16), w0_ref[...].astype(BF16),
                     preferred_element_type=F32).astype(BF16)
        s0_scr[...] = s0
        svt_scr[...] = s0.reshape(n // 128, 128, 128)
        out0_ref[...] = jnp.broadcast_to(b0_ref[...], out0_ref.shape)

    aw = a_ref[...].astype(BF16) * wv_ref[...]
    mv = jax.lax.dot_general(
        p_ref[...], aw, (((1,), (1,)), ((), ())),
        preferred_element_type=F32).astype(BF16)
    mv_ref[0, 0] = mv

    svt_j = svt_scr[pl.ds(j * TC, TC)]
    sb_i = s0_scr[pl.ds(i * BS, BS), :]
    a_side, b_side = _sides(mv, svt_j, sb_i)
    out0_ref[pl.ds(i * BS, BS), :] += a_side
    out0_ref[pl.ds(j * BS, BS), :] += b_side


def _pass2_body(mv_ref, out0_ref, w1_ref, b1_ref, gc_ref, s1_scr, svt_scr):
    i = pl.program_id(0)
    j = pl.program_id(1)
    n = gc_ref.shape[0]

    @pl.when(jnp.logical_and(i == 0, j == 0))
    def _init():
        s1 = (0.5 * jnp.dot(out0_ref[...].astype(BF16),
                            w1_ref[...].astype(BF16),
                            preferred_element_type=F32)).astype(BF16)
        s1_scr[...] = s1
        svt_scr[...] = s1.reshape(n // 128, 128, 128)
        gc_ref[...] = 0.5 * (out0_ref[...] +
                             jnp.broadcast_to(b1_ref[...], gc_ref.shape))

    mv = mv_ref[0, 0]
    svt_j = svt_scr[pl.ds(j * TC, TC)]
    sb_i = s1_scr[pl.ds(i * BS, BS), :]
    a_side, b_side = _sides(mv, svt_j, sb_i)
    gc_ref[pl.ds(i * BS, BS), :] += a_side
    gc_ref[pl.ds(j * BS, BS), :] += b_side


@jax.jit
def kernel(x_feature, all_adj_matrix, W0, b0, W1, b1, relation_weight):
    N, D_in = x_feature.shape
    D_out = W0.shape[1]
    n_rel = all_adj_matrix.shape[2]
    T = N // BS

    v = (all_adj_matrix.reshape(N, N // 128, 128, n_rel)
         .transpose(0, 1, 3, 2)
         .reshape(N, n_rel * N // 128, 128))

    wvals = relation_weight[jnp.arange(SC) % n_rel, 0]
    wv = jnp.broadcast_to(wvals[None, :, None], (1, SC, 128)).astype(BF16)

    pmat = (jnp.arange(SC)[None, :] // n_rel ==
            jnp.arange(TC)[:, None]).astype(BF16)

    b0r = b0.reshape(1, D_out)
    b1r = b1.reshape(1, D_out)

    common = dict(
        grid=(T, T),
        compiler_params=pltpu.CompilerParams(
            dimension_semantics=("arbitrary", "arbitrary")),
    )
    full = lambda r, c: pl.BlockSpec((r, c), lambda i, j: (0, 0))

    out0, m_v = pl.pallas_call(
        _pass1_body,
        in_specs=[
            pl.BlockSpec((BS, SC, 128), lambda i, j: (i, j, 0)),
            pl.BlockSpec((1, SC, 128), lambda i, j: (0, 0, 0)),
            pl.BlockSpec((TC, SC), lambda i, j: (0, 0)),
            full(N, D_in), full(D_in, D_out), full(1, D_out),
        ],
        out_specs=[
            pl.BlockSpec((N, D_out), lambda i, j: (0, 0)),
            pl.BlockSpec((1, 1, TC, BS, 128), lambda i, j: (i, j, 0, 0, 0)),
        ],
        out_shape=[
            jax.ShapeDtypeStruct((N, D_out), F32),
            jax.ShapeDtypeStruct((T, T, TC, BS, 128), BF16),
        ],
        scratch_shapes=[pltpu.VMEM((N, D_out), BF16),
                        pltpu.VMEM((N // 128, 128, 128), BF16)],
        **common,
    )(v, wv, pmat, x_feature, W0, b0r)

    gc = pl.pallas_call(
        _pass2_body,
        in_specs=[
            pl.BlockSpec((1, 1, TC, BS, 128), lambda i, j: (i, j, 0, 0, 0)),
            full(N, D_out), full(D_out, D_out), full(1, D_out),
        ],
        out_specs=pl.BlockSpec((N, D_out), lambda i, j: (0, 0)),
        out_shape=jax.ShapeDtypeStruct((N, D_out), F32),
        scratch_shapes=[pltpu.VMEM((N, D_out), BF16),
                        pltpu.VMEM((N // 128, 128, 128), BF16)],
        **common,
    )(m_v, out0, W1, b1r)
    return gc


# speedup vs baseline: 1.0333x; 1.0333x over previous
import jax
import jax.numpy as jnp
from jax.experimental import pallas as pl
from jax.experimental.pallas import tpu as pltpu

F32 = jnp.float32
BF16 = jnp.bfloat16


def _dup_planes_w(s2d, w0, w1, N):
    r3 = s2d.reshape(N // 128, 128, 128)
    pair = jnp.stack([w0 * r3, w1 * r3], axis=1)
    return pair.reshape(2 * N // 128, 128, 128)


def _sides(aw, sv_j, sb_i, wp, BCW):
    ga = jax.lax.dot_general(
        aw, sv_j, (((2,), (1,)), ((1,), (0,))),
        preferred_element_type=F32)
    a_side = jnp.sum(ga, axis=0)
    gb = jax.lax.dot_general(
        aw, sb_i, (((0,), (0,)), ((), ())),
        preferred_element_type=F32)
    sc = aw.shape[1]
    b_side = (gb.reshape(sc // 2, 2, 128, 128) * wp).sum(axis=1).reshape(
        BCW, 128)
    return a_side, b_side


def _make_body(second_pass, BR, BCW, SC):
    def body(a_ref, x_ref, w_ref, b_ref, rw_ref, out_ref, s_scr, sv_scr):
        i = pl.program_id(0)
        j = pl.program_id(1)
        n = out_ref.shape[0]
        w0 = rw_ref[0, 0]
        w1 = rw_ref[1, 0]

        @pl.when(jnp.logical_and(i == 0, j == 0))
        def _init():
            s = jnp.dot(x_ref[...].astype(BF16), w_ref[...].astype(BF16),
                        preferred_element_type=F32)
            if second_pass:
                s = 0.5 * s
            sb = s.astype(BF16)
            s_scr[...] = sb
            sv_scr[...] = _dup_planes_w(sb, w0.astype(BF16),
                                        w1.astype(BF16), n)
            bias = jnp.broadcast_to(b_ref[...], out_ref.shape)
            if second_pass:
                out_ref[...] = 0.5 * (x_ref[...] + bias)
            else:
                out_ref[...] = bias

        aw = a_ref[...].astype(BF16)
        sv_j = sv_scr[pl.ds(j * SC, SC)]
        sb_i = s_scr[pl.ds(i * BR, BR), :]
        wp = jnp.stack([w0, w1]).reshape(1, 2, 1, 1)
        a_side, b_side = _sides(aw, sv_j, sb_i, wp, BCW)
        out_ref[pl.ds(i * BR, BR), :] += a_side
        out_ref[pl.ds(j * BCW, BCW), :] += b_side
    return body


@jax.jit
def kernel(x_feature, all_adj_matrix, W0, b0, W1, b1, relation_weight):
    N, D_in = x_feature.shape
    D_out = W0.shape[1]
    n_rel = all_adj_matrix.shape[2]

    BR = N // 2
    BCW = max(N // 8, 128)
    SC = n_rel * BCW // 128
    v = (all_adj_matrix.reshape(N, N // 128, 128, n_rel)
         .transpose(0, 1, 3, 2)
         .reshape(N, n_rel * N // 128, 128))

    b0r = b0.reshape(1, D_out)
    b1r = b1.reshape(1, D_out)

    common = dict(
        grid=(N // BR, N // BCW),
        compiler_params=pltpu.CompilerParams(
            dimension_semantics=("arbitrary", "arbitrary")),
    )
    a_spec = pl.BlockSpec((BR, SC, 128), lambda i, j: (i, j, 0))
    full = lambda r, c: pl.BlockSpec((r, c), lambda i, j: (0, 0))

    out0 = pl.pallas_call(
        _make_body(False, BR, BCW, SC),
        in_specs=[a_spec, full(N, D_in), full(D_in, D_out), full(1, D_out),
                  full(n_rel, 1)],
        out_specs=pl.BlockSpec((N, D_out), lambda i, j: (0, 0)),
        out_shape=jax.ShapeDtypeStruct((N, D_out), F32),
        scratch_shapes=[pltpu.VMEM((N, D_out), BF16),
                        pltpu.VMEM((n_rel * N // 128, 128, 128), BF16)],
        **common,
    )(v, x_feature, W0, b0r, relation_weight)

    gc = pl.pallas_call(
        _make_body(True, BR, BCW, SC),
        in_specs=[a_spec, full(N, D_out), full(D_out, D_out), full(1, D_out),
                  full(n_rel, 1)],
        out_specs=pl.BlockSpec((N, D_out), lambda i, j: (0, 0)),
        out_shape=jax.ShapeDtypeStruct((N, D_out), F32),
        scratch_shapes=[pltpu.VMEM((N, D_out), BF16),
                        pltpu.VMEM((n_rel * N // 128, 128, 128), BF16)],
        **common,
    )(v, out0, W1, b1r, relation_weight)
    return gc
